# baseline (speedup 1.0000x reference)
import functools

import jax
import jax.numpy as jnp
from jax import lax
from jax.experimental import pallas as pl
from jax.experimental.pallas import tpu as pltpu
from jax.experimental.pallas import tpu_sc as plsc

_N = 1024
_D = 256
_M = 4096
_C = 512
_MM = 64
_K = 256

_NC = 2
_NS = 16
_NW = _NC * _NS
_CB = _C // _NW
_NB = _N // _NW

_mesh = plsc.VectorSubcoreMesh(core_axis_name="c", subcore_axis_name="s")


@functools.partial(
    pl.kernel,
    out_type=[
        jax.ShapeDtypeStruct((_C, _C), jnp.float32),
        jax.ShapeDtypeStruct((_N, 1), jnp.int32),
        jax.ShapeDtypeStruct((_N, _D), jnp.float32),
    ],
    mesh=_mesh,
    compiler_params=pltpu.CompilerParams(needs_layout_passes=False),
    scratch_types=[
        pltpu.VMEM((_M,), jnp.int32),
        pltpu.VMEM((_C,), jnp.int32),
        pltpu.VMEM((_K,), jnp.int32),
        pltpu.VMEM((_CB, _MM), jnp.int32),
        pltpu.VMEM((_CB, _MM), jnp.float32),
        pltpu.VMEM((_CB,), jnp.int32),
        pltpu.VMEM((_CB, _C), jnp.float32),
        pltpu.VMEM((_NB,), jnp.int32),
        pltpu.VMEM((_NB, 1), jnp.int32),
        pltpu.VMEM((_NB, _D), jnp.float32),
        pltpu.SemaphoreType.DMA,
        pltpu.SemaphoreType.DMA,
        pltpu.SemaphoreType.DMA,
    ],
)
def _sc_stage(n2c_hbm, c2n_hbm, ms_hbm, mn_hbm, cidx_hbm, nodes_hbm, e_hbm,
              a_hbm, cn_hbm, ce_hbm,
              n2c_v, inset_v, cidx_v, c2n_v, ms_v, mn_v, arow_v,
              nodes_v, cn_v, rows_v, sem, sem_in, sem_out):
    wid = lax.axis_index("s") * _NC + lax.axis_index("c")
    cbase = wid * _CB
    nbase = wid * _NB

    in_copies = [
        pltpu.async_copy(nodes_hbm.at[pl.ds(nbase, _NB)], nodes_v, sem_in),
        pltpu.async_copy(n2c_hbm, n2c_v, sem_in),
        pltpu.async_copy(cidx_hbm, cidx_v, sem_in),
        pltpu.async_copy(c2n_hbm.at[pl.ds(cbase, _CB)], c2n_v, sem_in),
        pltpu.async_copy(ms_hbm.at[pl.ds(cbase, _CB)], ms_v, sem_in),
        pltpu.async_copy(mn_hbm.at[pl.ds(cbase, _CB)], mn_v, sem_in),
    ]

    zi16 = jnp.zeros((16,), jnp.int32)
    zf16 = jnp.zeros((16,), jnp.float32)
    one16 = jnp.ones((16,), jnp.int32)
    iota16 = lax.iota(jnp.int32, 16)

    for i in range(_CB):
        for j in range(_C // 16):
            arow_v[i, pl.ds(j * 16, 16)] = zf16

    for cp in in_copies:
        cp.wait()

    gather = pltpu.async_copy(e_hbm.at[nodes_v], rows_v, sem)

    for i in range(_C // 16):
        inset_v[pl.ds(i * 16, 16)] = zi16
    for i in range(_K // 16):
        plsc.store_scatter(inset_v, [cidx_v[pl.ds(i * 16, 16)]], one16)

    mn16 = mn_v[pl.ds(0, _CB)]
    for m in range(_MM):
        col = jnp.full((16,), m, jnp.int32)
        members = plsc.load_gather(c2n_v, [iota16, col])
        neigh = plsc.load_gather(n2c_v, [members])
        inset = plsc.load_gather(inset_v, [neigh])
        keep = (mn16 > m) & (inset > 0)
        score = plsc.load_gather(ms_v, [iota16, col])
        w = jnp.where(keep, score, zf16)
        plsc.addupdate_scatter(arow_v, [iota16, neigh], w)

    for j in range(_NB // 16):
        nid = nodes_v[pl.ds(j * 16, 16)]
        cn = plsc.load_gather(n2c_v, [nid])
        flag = plsc.load_gather(inset_v, [cn])
        enc = cn + (1 - flag) * _C
        plsc.store_scatter(cn_v, [iota16 + (j * 16), zi16], enc)

    out_copies = [
        pltpu.async_copy(arow_v, a_hbm.at[pl.ds(cbase, _CB)], sem_out),
        pltpu.async_copy(cn_v, cn_hbm.at[pl.ds(nbase, _NB)], sem_out),
    ]
    gather.wait()
    out_copies.append(
        pltpu.async_copy(rows_v, ce_hbm.at[pl.ds(nbase, _NB)], sem_out))
    for cp in out_copies:
        cp.wait()


def _tc_pre_body(ne_ref, w1_ref, b1_ref, v1_ref, c1_ref, v2_ref, c2_ref,
                 hpre_ref, p2_ref):
    f32 = jnp.float32
    dot = functools.partial(jnp.dot, preferred_element_type=f32)
    hpre_ref[...] = dot(ne_ref[...], w1_ref[0:_D]) + b1_ref[...]
    h2 = jnp.maximum(dot(ne_ref[...], v1_ref[...]) + c1_ref[...], 0.0)
    p2_ref[...] = dot(h2, v2_ref[...]) + c2_ref[...]


_tc_pre = pl.pallas_call(
    _tc_pre_body,
    out_shape=[jax.ShapeDtypeStruct((_N, _D), jnp.float32),
               jax.ShapeDtypeStruct((_N, 1), jnp.float32)],
)


def _tc_body(a_ref, e_ref, cn_ref, ce_ref, hpre_ref, p2_ref,
             w1_ref, w2_ref, b2_ref, o_ref):
    f32 = jnp.float32
    dot = functools.partial(jnp.dot, preferred_element_type=f32)

    comm_emb = dot(a_ref[...], e_ref[...])
    iota = lax.broadcasted_iota(jnp.int32, (_N, _C), 1)
    onehot = (cn_ref[...] == iota).astype(f32)
    member = dot(onehot, comm_emb)

    w1 = w1_ref[...]
    h = (hpre_ref[...] + dot(ce_ref[...], w1[0:_D])
         + dot(member, w1[_D:2 * _D]))
    h = jnp.maximum(h, 0.0)
    p1 = dot(h, w2_ref[...]) + b2_ref[...]

    o_ref[...] = jnp.where(cn_ref[...] < _C, p1, p2_ref[...])


_tc_stage = pl.pallas_call(
    _tc_body,
    grid=(1,),
    in_specs=[
        pl.BlockSpec((_C, _C), lambda i: (0, 0)),
        pl.BlockSpec((_C, _D), lambda i: (0, 0)),
        pl.BlockSpec((_N, 1), lambda i: (0, 0)),
        pl.BlockSpec((_N, _D), lambda i: (0, 0)),
        pl.BlockSpec((_N, _D), lambda i: (0, 0)),
        pl.BlockSpec((_N, 1), lambda i: (0, 0)),
        pl.BlockSpec((2 * _D, _D), lambda i: (0, 0)),
        pl.BlockSpec((_D, 1), lambda i: (0, 0)),
        pl.BlockSpec((1, 1), lambda i: (0, 0)),
    ],
    out_shape=jax.ShapeDtypeStruct((_N, 1), jnp.float32),
    out_specs=pl.BlockSpec((_N, 1), lambda i: (0, 0)),
)


def kernel(node_emb, node2community, community2node, member_score, member_num,
           community_embeddings, community_index, nodes,
           W1, b1, W2, b2, V1, c1, V2, c2):
    a_mat, cn, ce = _sc_stage(
        node2community, community2node, member_score, member_num,
        community_index, nodes, community_embeddings)

    hpre, p2 = _tc_pre(node_emb, W1, b1.reshape(1, _D),
                       V1, c1.reshape(1, _D // 2), V2, c2.reshape(1, 1))

    pred = _tc_stage(
        a_mat, community_embeddings, cn, ce, hpre, p2,
        W1[_D:], W2, b2.reshape(1, 1))
    return pred.reshape(_N)


# speedup vs baseline: 1.0639x; 1.0639x over previous
import functools

import jax
import jax.numpy as jnp
from jax import lax
from jax.experimental import pallas as pl
from jax.experimental.pallas import tpu as pltpu
from jax.experimental.pallas import tpu_sc as plsc

_N = 1024
_D = 256
_M = 4096
_C = 512
_MM = 64
_K = 256

_NC = 2
_NS = 16
_NW = _NC * _NS
_CB = _C // _NW
_NB = _N // _NW

_mesh = plsc.VectorSubcoreMesh(core_axis_name="c", subcore_axis_name="s")


@functools.partial(
    pl.kernel,
    out_type=[
        jax.ShapeDtypeStruct((_C, _C), jnp.float32),
        jax.ShapeDtypeStruct((_N, 1), jnp.int32),
        jax.ShapeDtypeStruct((_N, _D), jnp.float32),
    ],
    mesh=_mesh,
    compiler_params=pltpu.CompilerParams(needs_layout_passes=False),
    scratch_types=[
        pltpu.VMEM((_M,), jnp.int32),
        pltpu.VMEM((_C,), jnp.int32),
        pltpu.VMEM((_K,), jnp.int32),
        pltpu.VMEM((_CB, _MM), jnp.int32),
        pltpu.VMEM((_CB, _MM), jnp.float32),
        pltpu.VMEM((_CB,), jnp.int32),
        pltpu.VMEM((_CB, _C), jnp.float32),
        pltpu.VMEM((_NB,), jnp.int32),
        pltpu.VMEM((_NB, 1), jnp.int32),
        pltpu.VMEM((_NB, _D), jnp.float32),
        pltpu.SemaphoreType.DMA,
        pltpu.SemaphoreType.DMA,
        pltpu.SemaphoreType.DMA,
    ],
)
def _sc_stage(n2c_hbm, c2n_hbm, ms_hbm, mn_hbm, cidx_hbm, nodes_hbm, e_hbm,
              a_hbm, cn_hbm, ce_hbm,
              n2c_v, inset_v, cidx_v, c2n_v, ms_v, mn_v, arow_v,
              nodes_v, cn_v, rows_v, sem, sem_in, sem_out):
    wid = lax.axis_index("s") * _NC + lax.axis_index("c")
    cbase = wid * _CB
    nbase = wid * _NB

    in_copies = [
        pltpu.async_copy(nodes_hbm.at[pl.ds(nbase, _NB)], nodes_v, sem_in),
        pltpu.async_copy(n2c_hbm, n2c_v, sem_in),
        pltpu.async_copy(cidx_hbm, cidx_v, sem_in),
        pltpu.async_copy(c2n_hbm.at[pl.ds(cbase, _CB)], c2n_v, sem_in),
        pltpu.async_copy(ms_hbm.at[pl.ds(cbase, _CB)], ms_v, sem_in),
        pltpu.async_copy(mn_hbm.at[pl.ds(cbase, _CB)], mn_v, sem_in),
    ]

    zi16 = jnp.zeros((16,), jnp.int32)
    zf16 = jnp.zeros((16,), jnp.float32)
    one16 = jnp.ones((16,), jnp.int32)
    iota16 = lax.iota(jnp.int32, 16)

    for i in range(_CB):
        for j in range(_C // 16):
            arow_v[i, pl.ds(j * 16, 16)] = zf16

    for cp in in_copies:
        cp.wait()

    gather = pltpu.async_copy(e_hbm.at[nodes_v], rows_v, sem)

    for i in range(_C // 16):
        inset_v[pl.ds(i * 16, 16)] = zi16
    for i in range(_K // 16):
        plsc.store_scatter(inset_v, [cidx_v[pl.ds(i * 16, 16)]], one16)

    mn16 = mn_v[pl.ds(0, _CB)]

    @plsc.parallel_loop(0, _MM, 1, unroll=8)
    def _mbody(m):
        col = jnp.zeros((16,), jnp.int32) + m
        members = plsc.load_gather(c2n_v, [iota16, col])
        neigh = plsc.load_gather(n2c_v, [members])
        inset = plsc.load_gather(inset_v, [neigh])
        keep = (mn16 > m) & (inset > 0)
        score = plsc.load_gather(ms_v, [iota16, col])
        w = jnp.where(keep, score, zf16)
        plsc.addupdate_scatter(arow_v, [iota16, neigh], w)

    for j in range(_NB // 16):
        nid = nodes_v[pl.ds(j * 16, 16)]
        cn = plsc.load_gather(n2c_v, [nid])
        flag = plsc.load_gather(inset_v, [cn])
        enc = cn + (1 - flag) * _C
        plsc.store_scatter(cn_v, [iota16 + (j * 16), zi16], enc)

    out_copies = [
        pltpu.async_copy(arow_v, a_hbm.at[pl.ds(cbase, _CB)], sem_out),
        pltpu.async_copy(cn_v, cn_hbm.at[pl.ds(nbase, _NB)], sem_out),
    ]
    gather.wait()
    out_copies.append(
        pltpu.async_copy(rows_v, ce_hbm.at[pl.ds(nbase, _NB)], sem_out))
    for cp in out_copies:
        cp.wait()


def _tc_body(a_ref, e_ref, cn_ref, ce_ref, ne_ref,
             w1_ref, b1_ref, w2_ref, b2_ref, v1_ref, c1_ref, v2_ref, c2_ref,
             o_ref):
    f32 = jnp.float32
    dot = functools.partial(jnp.dot, preferred_element_type=f32)

    comm_emb = dot(a_ref[...], e_ref[...])
    iota = lax.broadcasted_iota(jnp.int32, (_N, _C), 1)
    onehot = (cn_ref[...] == iota).astype(f32)
    member = dot(onehot, comm_emb)

    w1 = w1_ref[...]
    h = (dot(ne_ref[...], w1[0:_D]) + dot(ce_ref[...], w1[_D:2 * _D])
         + dot(member, w1[2 * _D:3 * _D]) + b1_ref[...])
    h = jnp.maximum(h, 0.0)
    p1 = dot(h, w2_ref[...]) + b2_ref[...]

    h2 = jnp.maximum(dot(ne_ref[...], v1_ref[...]) + c1_ref[...], 0.0)
    p2 = dot(h2, v2_ref[...]) + c2_ref[...]

    o_ref[...] = jnp.where(cn_ref[...] < _C, p1, p2)


_tc_stage = pl.pallas_call(
    _tc_body,
    grid=(1,),
    in_specs=[
        pl.BlockSpec((_C, _C), lambda i: (0, 0)),
        pl.BlockSpec((_C, _D), lambda i: (0, 0)),
        pl.BlockSpec((_N, 1), lambda i: (0, 0)),
        pl.BlockSpec((_N, _D), lambda i: (0, 0)),
        pl.BlockSpec((_N, _D), lambda i: (0, 0)),
        pl.BlockSpec((3 * _D, _D), lambda i: (0, 0)),
        pl.BlockSpec((1, _D), lambda i: (0, 0)),
        pl.BlockSpec((_D, 1), lambda i: (0, 0)),
        pl.BlockSpec((1, 1), lambda i: (0, 0)),
        pl.BlockSpec((_D, _D // 2), lambda i: (0, 0)),
        pl.BlockSpec((1, _D // 2), lambda i: (0, 0)),
        pl.BlockSpec((_D // 2, 1), lambda i: (0, 0)),
        pl.BlockSpec((1, 1), lambda i: (0, 0)),
    ],
    out_shape=jax.ShapeDtypeStruct((_N, 1), jnp.float32),
    out_specs=pl.BlockSpec((_N, 1), lambda i: (0, 0)),
)


def kernel(node_emb, node2community, community2node, member_score, member_num,
           community_embeddings, community_index, nodes,
           W1, b1, W2, b2, V1, c1, V2, c2):
    a_mat, cn, ce = _sc_stage(
        node2community, community2node, member_score, member_num,
        community_index, nodes, community_embeddings)

    pred = _tc_stage(
        a_mat, community_embeddings,
        cn, ce, node_emb,
        W1, b1.reshape(1, _D), W2, b2.reshape(1, 1),
        V1, c1.reshape(1, _D // 2), V2, c2.reshape(1, 1))
    return pred.reshape(_N)


# speedup vs baseline: 1.0753x; 1.0107x over previous
import functools

import jax
import jax.numpy as jnp
from jax import lax
from jax.experimental import pallas as pl
from jax.experimental.pallas import tpu as pltpu
from jax.experimental.pallas import tpu_sc as plsc

_N = 1024
_D = 256
_M = 4096
_C = 512
_MM = 64
_K = 256

_NC = 2
_NS = 16
_NW = _NC * _NS
_CB = _C // _NW
_NB = _N // _NW

_mesh = plsc.VectorSubcoreMesh(core_axis_name="c", subcore_axis_name="s")


@functools.partial(
    pl.kernel,
    out_type=[
        jax.ShapeDtypeStruct((_C, _C), jnp.float32),
        jax.ShapeDtypeStruct((_N, 1), jnp.int32),
        jax.ShapeDtypeStruct((_N, _D), jnp.float32),
    ],
    mesh=_mesh,
    compiler_params=pltpu.CompilerParams(needs_layout_passes=False),
    scratch_types=[
        pltpu.VMEM((_M,), jnp.int32),
        pltpu.VMEM((_C,), jnp.int32),
        pltpu.VMEM((_K,), jnp.int32),
        pltpu.VMEM((_CB, _MM), jnp.int32),
        pltpu.VMEM((_CB, _MM), jnp.float32),
        pltpu.VMEM((_CB,), jnp.int32),
        pltpu.VMEM((_CB, _C), jnp.float32),
        pltpu.VMEM((_NB,), jnp.int32),
        pltpu.VMEM((_NB, 1), jnp.int32),
        pltpu.VMEM((_NB, _D), jnp.float32),
        pltpu.SemaphoreType.DMA,
        pltpu.SemaphoreType.DMA,
        pltpu.SemaphoreType.DMA,
    ],
)
def _sc_stage(n2c_hbm, c2n_hbm, ms_hbm, mn_hbm, cidx_hbm, nodes_hbm, e_hbm,
              a_hbm, cn_hbm, ce_hbm,
              n2c_v, inset_v, cidx_v, c2n_v, ms_v, mn_v, arow_v,
              nodes_v, cn_v, rows_v, sem, sem_in, sem_out):
    wid = lax.axis_index("s") * _NC + lax.axis_index("c")
    cbase = wid * _CB
    nbase = wid * _NB

    in_copies = [
        pltpu.async_copy(nodes_hbm.at[pl.ds(nbase, _NB)], nodes_v, sem_in),
        pltpu.async_copy(n2c_hbm, n2c_v, sem_in),
        pltpu.async_copy(cidx_hbm, cidx_v, sem_in),
        pltpu.async_copy(c2n_hbm.at[pl.ds(cbase, _CB)], c2n_v, sem_in),
        pltpu.async_copy(ms_hbm.at[pl.ds(cbase, _CB)], ms_v, sem_in),
        pltpu.async_copy(mn_hbm.at[pl.ds(cbase, _CB)], mn_v, sem_in),
    ]

    zi16 = jnp.zeros((16,), jnp.int32)
    zf16 = jnp.zeros((16,), jnp.float32)
    one16 = jnp.ones((16,), jnp.int32)
    iota16 = lax.iota(jnp.int32, 16)

    @plsc.parallel_loop(0, _CB, 1, unroll=2)
    def _zbody(i):
        for j in range(_C // 16):
            arow_v[i, pl.ds(j * 16, 16)] = zf16

    for i in range(_C // 16):
        inset_v[pl.ds(i * 16, 16)] = zi16

    for cp in in_copies:
        cp.wait()

    gather = pltpu.async_copy(e_hbm.at[nodes_v], rows_v, sem)

    for i in range(_K // 16):
        plsc.store_scatter(inset_v, [cidx_v[pl.ds(i * 16, 16)]], one16)

    mn16 = mn_v[pl.ds(0, _CB)]

    @plsc.parallel_loop(0, _MM, 1, unroll=16)
    def _mbody(m):
        col = jnp.zeros((16,), jnp.int32) + m
        members = plsc.load_gather(c2n_v, [iota16, col])
        neigh = plsc.load_gather(n2c_v, [members])
        inset = plsc.load_gather(inset_v, [neigh])
        keep = (mn16 > m) & (inset > 0)
        score = plsc.load_gather(ms_v, [iota16, col])
        w = jnp.where(keep, score, zf16)
        plsc.addupdate_scatter(arow_v, [iota16, neigh], w)

    for j in range(_NB // 16):
        nid = nodes_v[pl.ds(j * 16, 16)]
        cn = plsc.load_gather(n2c_v, [nid])
        flag = plsc.load_gather(inset_v, [cn])
        enc = cn + (1 - flag) * _C
        plsc.store_scatter(cn_v, [iota16 + (j * 16), zi16], enc)

    out_copies = [
        pltpu.async_copy(arow_v, a_hbm.at[pl.ds(cbase, _CB)], sem_out),
        pltpu.async_copy(cn_v, cn_hbm.at[pl.ds(nbase, _NB)], sem_out),
    ]
    gather.wait()
    out_copies.append(
        pltpu.async_copy(rows_v, ce_hbm.at[pl.ds(nbase, _NB)], sem_out))
    for cp in out_copies:
        cp.wait()


def _tc_body(a_ref, e_ref, cn_ref, ce_ref, ne_ref,
             w1_ref, b1_ref, w2_ref, b2_ref, v1_ref, c1_ref, v2_ref, c2_ref,
             o_ref):
    f32 = jnp.float32
    dot = functools.partial(jnp.dot, preferred_element_type=f32)

    comm_emb = dot(a_ref[...], e_ref[...])
    iota = lax.broadcasted_iota(jnp.int32, (_N, _C), 1)
    onehot = (cn_ref[...] == iota).astype(f32)
    member = dot(onehot, comm_emb)

    w1 = w1_ref[...]
    h = (dot(ne_ref[...], w1[0:_D]) + dot(ce_ref[...], w1[_D:2 * _D])
         + dot(member, w1[2 * _D:3 * _D]) + b1_ref[...])
    h = jnp.maximum(h, 0.0)
    p1 = dot(h, w2_ref[...]) + b2_ref[...]

    h2 = jnp.maximum(dot(ne_ref[...], v1_ref[...]) + c1_ref[...], 0.0)
    p2 = dot(h2, v2_ref[...]) + c2_ref[...]

    o_ref[...] = jnp.where(cn_ref[...] < _C, p1, p2)


_tc_stage = pl.pallas_call(
    _tc_body,
    grid=(1,),
    in_specs=[
        pl.BlockSpec((_C, _C), lambda i: (0, 0)),
        pl.BlockSpec((_C, _D), lambda i: (0, 0)),
        pl.BlockSpec((_N, 1), lambda i: (0, 0)),
        pl.BlockSpec((_N, _D), lambda i: (0, 0)),
        pl.BlockSpec((_N, _D), lambda i: (0, 0)),
        pl.BlockSpec((3 * _D, _D), lambda i: (0, 0)),
        pl.BlockSpec((1, _D), lambda i: (0, 0)),
        pl.BlockSpec((_D, 1), lambda i: (0, 0)),
        pl.BlockSpec((1, 1), lambda i: (0, 0)),
        pl.BlockSpec((_D, _D // 2), lambda i: (0, 0)),
        pl.BlockSpec((1, _D // 2), lambda i: (0, 0)),
        pl.BlockSpec((_D // 2, 1), lambda i: (0, 0)),
        pl.BlockSpec((1, 1), lambda i: (0, 0)),
    ],
    out_shape=jax.ShapeDtypeStruct((_N, 1), jnp.float32),
    out_specs=pl.BlockSpec((_N, 1), lambda i: (0, 0)),
)


def kernel(node_emb, node2community, community2node, member_score, member_num,
           community_embeddings, community_index, nodes,
           W1, b1, W2, b2, V1, c1, V2, c2):
    a_mat, cn, ce = _sc_stage(
        node2community, community2node, member_score, member_num,
        community_index, nodes, community_embeddings)

    pred = _tc_stage(
        a_mat, community_embeddings,
        cn, ce, node_emb,
        W1, b1.reshape(1, _D), W2, b2.reshape(1, 1),
        V1, c1.reshape(1, _D // 2), V2, c2.reshape(1, 1))
    return pred.reshape(_N)


# speedup vs baseline: 1.0754x; 1.0001x over previous
import functools

import jax
import jax.numpy as jnp
from jax import lax
from jax.experimental import pallas as pl
from jax.experimental.pallas import tpu as pltpu
from jax.experimental.pallas import tpu_sc as plsc

_N = 1024
_D = 256
_M = 4096
_C = 512
_MM = 64
_K = 256

_NC = 2
_NS = 16
_NW = _NC * _NS
_CB = _C // _NW
_NB = _N // _NW

_mesh = plsc.VectorSubcoreMesh(core_axis_name="c", subcore_axis_name="s")


@functools.partial(
    pl.kernel,
    out_type=[
        jax.ShapeDtypeStruct((_C, _C), jnp.float32),
        jax.ShapeDtypeStruct((_N, 1), jnp.int32),
        jax.ShapeDtypeStruct((_N, _D), jnp.float32),
    ],
    mesh=_mesh,
    compiler_params=pltpu.CompilerParams(needs_layout_passes=False, disable_bounds_checks=True, disable_semaphore_checks=True),
    scratch_types=[
        pltpu.VMEM((_M,), jnp.int32),
        pltpu.VMEM((_C,), jnp.int32),
        pltpu.VMEM((_K,), jnp.int32),
        pltpu.VMEM((_CB, _MM), jnp.int32),
        pltpu.VMEM((_CB, _MM), jnp.float32),
        pltpu.VMEM((_CB,), jnp.int32),
        pltpu.VMEM((_CB, _C), jnp.float32),
        pltpu.VMEM((_NB,), jnp.int32),
        pltpu.VMEM((_NB, 1), jnp.int32),
        pltpu.VMEM((_NB, _D), jnp.float32),
        pltpu.SemaphoreType.DMA,
        pltpu.SemaphoreType.DMA,
        pltpu.SemaphoreType.DMA,
    ],
)
def _sc_stage(n2c_hbm, c2n_hbm, ms_hbm, mn_hbm, cidx_hbm, nodes_hbm, e_hbm,
              a_hbm, cn_hbm, ce_hbm,
              n2c_v, inset_v, cidx_v, c2n_v, ms_v, mn_v, arow_v,
              nodes_v, cn_v, rows_v, sem, sem_in, sem_out):
    wid = lax.axis_index("s") * _NC + lax.axis_index("c")
    cbase = wid * _CB
    nbase = wid * _NB

    in_copies = [
        pltpu.async_copy(nodes_hbm.at[pl.ds(nbase, _NB)], nodes_v, sem_in),
        pltpu.async_copy(n2c_hbm, n2c_v, sem_in),
        pltpu.async_copy(cidx_hbm, cidx_v, sem_in),
        pltpu.async_copy(c2n_hbm.at[pl.ds(cbase, _CB)], c2n_v, sem_in),
        pltpu.async_copy(ms_hbm.at[pl.ds(cbase, _CB)], ms_v, sem_in),
        pltpu.async_copy(mn_hbm.at[pl.ds(cbase, _CB)], mn_v, sem_in),
    ]

    zi16 = jnp.zeros((16,), jnp.int32)
    zf16 = jnp.zeros((16,), jnp.float32)
    one16 = jnp.ones((16,), jnp.int32)
    iota16 = lax.iota(jnp.int32, 16)

    @plsc.parallel_loop(0, _CB, 1, unroll=2)
    def _zbody(i):
        for j in range(_C // 16):
            arow_v[i, pl.ds(j * 16, 16)] = zf16

    for i in range(_C // 16):
        inset_v[pl.ds(i * 16, 16)] = zi16

    for cp in in_copies:
        cp.wait()

    gather = pltpu.async_copy(e_hbm.at[nodes_v], rows_v, sem)

    for i in range(_K // 16):
        plsc.store_scatter(inset_v, [cidx_v[pl.ds(i * 16, 16)]], one16)

    mn16 = mn_v[pl.ds(0, _CB)]

    @plsc.parallel_loop(0, _MM, 1, unroll=16)
    def _mbody(m):
        col = jnp.zeros((16,), jnp.int32) + m
        members = plsc.load_gather(c2n_v, [iota16, col])
        neigh = plsc.load_gather(n2c_v, [members])
        inset = plsc.load_gather(inset_v, [neigh])
        keep = (mn16 > m) & (inset > 0)
        score = plsc.load_gather(ms_v, [iota16, col])
        w = jnp.where(keep, score, zf16)
        plsc.addupdate_scatter(arow_v, [iota16, neigh], w)

    for j in range(_NB // 16):
        nid = nodes_v[pl.ds(j * 16, 16)]
        cn = plsc.load_gather(n2c_v, [nid])
        flag = plsc.load_gather(inset_v, [cn])
        enc = cn + (1 - flag) * _C
        plsc.store_scatter(cn_v, [iota16 + (j * 16), zi16], enc)

    out_copies = [
        pltpu.async_copy(arow_v, a_hbm.at[pl.ds(cbase, _CB)], sem_out),
        pltpu.async_copy(cn_v, cn_hbm.at[pl.ds(nbase, _NB)], sem_out),
    ]
    gather.wait()
    out_copies.append(
        pltpu.async_copy(rows_v, ce_hbm.at[pl.ds(nbase, _NB)], sem_out))
    for cp in out_copies:
        cp.wait()


def _tc_body(a_ref, e_ref, cn_ref, ce_ref, ne_ref,
             w1_ref, b1_ref, w2_ref, b2_ref, v1_ref, c1_ref, v2_ref, c2_ref,
             o_ref):
    f32 = jnp.float32
    dot = functools.partial(jnp.dot, preferred_element_type=f32)

    comm_emb = dot(a_ref[...], e_ref[...])
    iota = lax.broadcasted_iota(jnp.int32, (_N, _C), 1)
    onehot = (cn_ref[...] == iota).astype(f32)
    member = dot(onehot, comm_emb)

    w1 = w1_ref[...]
    h = (dot(ne_ref[...], w1[0:_D]) + dot(ce_ref[...], w1[_D:2 * _D])
         + dot(member, w1[2 * _D:3 * _D]) + b1_ref[...])
    h = jnp.maximum(h, 0.0)
    p1 = dot(h, w2_ref[...]) + b2_ref[...]

    h2 = jnp.maximum(dot(ne_ref[...], v1_ref[...]) + c1_ref[...], 0.0)
    p2 = dot(h2, v2_ref[...]) + c2_ref[...]

    o_ref[...] = jnp.where(cn_ref[...] < _C, p1, p2)


_tc_stage = pl.pallas_call(
    _tc_body,
    grid=(1,),
    in_specs=[
        pl.BlockSpec((_C, _C), lambda i: (0, 0)),
        pl.BlockSpec((_C, _D), lambda i: (0, 0)),
        pl.BlockSpec((_N, 1), lambda i: (0, 0)),
        pl.BlockSpec((_N, _D), lambda i: (0, 0)),
        pl.BlockSpec((_N, _D), lambda i: (0, 0)),
        pl.BlockSpec((3 * _D, _D), lambda i: (0, 0)),
        pl.BlockSpec((1, _D), lambda i: (0, 0)),
        pl.BlockSpec((_D, 1), lambda i: (0, 0)),
        pl.BlockSpec((1, 1), lambda i: (0, 0)),
        pl.BlockSpec((_D, _D // 2), lambda i: (0, 0)),
        pl.BlockSpec((1, _D // 2), lambda i: (0, 0)),
        pl.BlockSpec((_D // 2, 1), lambda i: (0, 0)),
        pl.BlockSpec((1, 1), lambda i: (0, 0)),
    ],
    out_shape=jax.ShapeDtypeStruct((_N, 1), jnp.float32),
    out_specs=pl.BlockSpec((_N, 1), lambda i: (0, 0)),
)


def kernel(node_emb, node2community, community2node, member_score, member_num,
           community_embeddings, community_index, nodes,
           W1, b1, W2, b2, V1, c1, V2, c2):
    a_mat, cn, ce = _sc_stage(
        node2community, community2node, member_score, member_num,
        community_index, nodes, community_embeddings)

    pred = _tc_stage(
        a_mat, community_embeddings,
        cn, ce, node_emb,
        W1, b1.reshape(1, _D), W2, b2.reshape(1, 1),
        V1, c1.reshape(1, _D // 2), V2, c2.reshape(1, 1))
    return pred.reshape(_N)


# speedup vs baseline: 1.0755x; 1.0001x over previous
import functools

import jax
import jax.numpy as jnp
from jax import lax
from jax.experimental import pallas as pl
from jax.experimental.pallas import tpu as pltpu
from jax.experimental.pallas import tpu_sc as plsc

_N = 1024
_D = 256
_M = 4096
_C = 512
_MM = 64
_K = 256

_NC = 2
_NS = 16
_NW = _NC * _NS
_CB = _C // _NW
_NB = _N // _NW

_mesh = plsc.VectorSubcoreMesh(core_axis_name="c", subcore_axis_name="s")


@functools.partial(
    pl.kernel,
    out_type=[
        jax.ShapeDtypeStruct((_C, _C), jnp.float32),
        jax.ShapeDtypeStruct((_N, 1), jnp.int32),
        jax.ShapeDtypeStruct((_N, _D), jnp.float32),
    ],
    mesh=_mesh,
    compiler_params=pltpu.CompilerParams(needs_layout_passes=False),
    scratch_types=[
        pltpu.VMEM((_M,), jnp.int32),
        pltpu.VMEM((_C,), jnp.int32),
        pltpu.VMEM((_K,), jnp.int32),
        pltpu.VMEM((_CB, _MM), jnp.int32),
        pltpu.VMEM((_CB, _MM), jnp.float32),
        pltpu.VMEM((_CB,), jnp.int32),
        pltpu.VMEM((_CB, _C), jnp.float32),
        pltpu.VMEM((_NB,), jnp.int32),
        pltpu.VMEM((_NB, 1), jnp.int32),
        pltpu.VMEM((_NB, _D), jnp.float32),
        pltpu.SemaphoreType.DMA,
        pltpu.SemaphoreType.DMA,
        pltpu.SemaphoreType.DMA,
    ],
)
def _sc_stage(n2c_hbm, c2n_hbm, ms_hbm, mn_hbm, cidx_hbm, nodes_hbm, e_hbm,
              a_hbm, cn_hbm, ce_hbm,
              n2c_v, inset_v, cidx_v, c2n_v, ms_v, mn_v, arow_v,
              nodes_v, cn_v, rows_v, sem, sem_in, sem_out):
    wid = lax.axis_index("s") * _NC + lax.axis_index("c")
    cbase = wid * _CB
    nbase = wid * _NB

    in_copies = [
        pltpu.async_copy(nodes_hbm.at[pl.ds(nbase, _NB)], nodes_v, sem_in),
        pltpu.async_copy(n2c_hbm, n2c_v, sem_in),
        pltpu.async_copy(cidx_hbm, cidx_v, sem_in),
        pltpu.async_copy(c2n_hbm.at[pl.ds(cbase, _CB)], c2n_v, sem_in),
        pltpu.async_copy(ms_hbm.at[pl.ds(cbase, _CB)], ms_v, sem_in),
        pltpu.async_copy(mn_hbm.at[pl.ds(cbase, _CB)], mn_v, sem_in),
    ]

    zi16 = jnp.zeros((16,), jnp.int32)
    zf16 = jnp.zeros((16,), jnp.float32)
    one16 = jnp.ones((16,), jnp.int32)
    iota16 = lax.iota(jnp.int32, 16)

    @plsc.parallel_loop(0, _CB, 1, unroll=2)
    def _zbody(i):
        for j in range(_C // 16):
            arow_v[i, pl.ds(j * 16, 16)] = zf16

    for i in range(_C // 16):
        inset_v[pl.ds(i * 16, 16)] = zi16

    for cp in in_copies:
        cp.wait()

    gather = pltpu.async_copy(e_hbm.at[nodes_v], rows_v, sem)

    for i in range(_K // 16):
        plsc.store_scatter(inset_v, [cidx_v[pl.ds(i * 16, 16)]], one16)

    mn16 = mn_v[pl.ds(0, _CB)]

    @plsc.parallel_loop(0, _MM, 1, unroll=16)
    def _mbody(m):
        col = jnp.zeros((16,), jnp.int32) + m
        members = plsc.load_gather(c2n_v, [iota16, col])
        neigh = plsc.load_gather(n2c_v, [members])
        inset = plsc.load_gather(inset_v, [neigh])
        keep = (mn16 > m) & (inset > 0)
        score = plsc.load_gather(ms_v, [iota16, col])
        w = jnp.where(keep, score, zf16)
        plsc.addupdate_scatter(arow_v, [iota16, neigh], w)

    for j in range(_NB // 16):
        nid = nodes_v[pl.ds(j * 16, 16)]
        cn = plsc.load_gather(n2c_v, [nid])
        flag = plsc.load_gather(inset_v, [cn])
        enc = cn + (1 - flag) * _C
        plsc.store_scatter(cn_v, [iota16 + (j * 16), zi16], enc)

    out_copies = [
        pltpu.async_copy(arow_v, a_hbm.at[pl.ds(cbase, _CB)], sem_out),
        pltpu.async_copy(cn_v, cn_hbm.at[pl.ds(nbase, _NB)], sem_out),
    ]
    gather.wait()
    out_copies.append(
        pltpu.async_copy(rows_v, ce_hbm.at[pl.ds(nbase, _NB)], sem_out))
    for cp in out_copies:
        cp.wait()


def _tc_body(a_ref, e_ref, cn_ref, ce_ref, ne_ref,
             w1_ref, b1_ref, w2_ref, b2_ref, v1_ref, c1_ref, v2_ref, c2_ref,
             o_ref):
    f32 = jnp.float32
    dot = functools.partial(jnp.dot, preferred_element_type=f32)

    comm_emb = dot(a_ref[...], e_ref[...])
    iota = lax.broadcasted_iota(jnp.int32, (_N, _C), 1)
    onehot = (cn_ref[...] == iota).astype(f32)
    member = dot(onehot, comm_emb)

    w1 = w1_ref[...]
    h = (dot(ne_ref[...], w1[0:_D]) + dot(ce_ref[...], w1[_D:2 * _D])
         + dot(member, w1[2 * _D:3 * _D]) + b1_ref[...])
    h = jnp.maximum(h, 0.0)
    p1 = dot(h, w2_ref[...]) + b2_ref[...]

    h2 = jnp.maximum(dot(ne_ref[...], v1_ref[...]) + c1_ref[...], 0.0)
    p2 = dot(h2, v2_ref[...]) + c2_ref[...]

    o_ref[...] = jnp.where(cn_ref[...] < _C, p1, p2)


_tc_stage = pl.pallas_call(
    _tc_body,
    grid=(1,),
    in_specs=[
        pl.BlockSpec((_C, _C), lambda i: (0, 0)),
        pl.BlockSpec((_C, _D), lambda i: (0, 0)),
        pl.BlockSpec((_N, 1), lambda i: (0, 0)),
        pl.BlockSpec((_N, _D), lambda i: (0, 0)),
        pl.BlockSpec((_N, _D), lambda i: (0, 0)),
        pl.BlockSpec((3 * _D, _D), lambda i: (0, 0)),
        pl.BlockSpec((1, _D), lambda i: (0, 0)),
        pl.BlockSpec((_D, 1), lambda i: (0, 0)),
        pl.BlockSpec((1, 1), lambda i: (0, 0)),
        pl.BlockSpec((_D, _D // 2), lambda i: (0, 0)),
        pl.BlockSpec((1, _D // 2), lambda i: (0, 0)),
        pl.BlockSpec((_D // 2, 1), lambda i: (0, 0)),
        pl.BlockSpec((1, 1), lambda i: (0, 0)),
    ],
    out_shape=jax.ShapeDtypeStruct((_N, 1), jnp.float32),
    out_specs=pl.BlockSpec((_N, 1), lambda i: (0, 0)),
)


def kernel(node_emb, node2community, community2node, member_score, member_num,
           community_embeddings, community_index, nodes,
           W1, b1, W2, b2, V1, c1, V2, c2):
    a_mat, cn, ce = _sc_stage(
        node2community, community2node, member_score, member_num,
        community_index, nodes, community_embeddings)

    pred = _tc_stage(
        a_mat, community_embeddings,
        cn, ce, node_emb,
        W1, b1.reshape(1, _D), W2, b2.reshape(1, 1),
        V1, c1.reshape(1, _D // 2), V2, c2.reshape(1, 1))
    return pred.reshape(_N)
